# baseline (device time: 259297 ns/iter reference)
import jax
import jax.numpy as jnp
from jax import lax
from jax.experimental import pallas as pl
from jax.experimental.pallas import tpu as pltpu

CH = 64


def kernel(x, dest):
    t, d = x.shape
    maxc = t // CH

    iota = jnp.arange(t, dtype=jnp.int32)
    d0 = dest == 0
    cum0 = jnp.cumsum(d0.astype(jnp.int32))
    c0 = cum0[-1]
    rank = jnp.where(d0, cum0 - 1, iota + c0 - cum0)
    cols = iota[None, :]
    rows = iota[:, None]
    inv = jnp.sum(jnp.where(rank[:, None] == cols, rows, 0), axis=0)
    zpad = jnp.zeros((CH,), jnp.int32)
    idxp = jnp.concatenate([zpad, inv, zpad, zpad]).astype(jnp.int32)
    xs_pad = x[idxp]
    cnt = jnp.reshape(c0, (1,))

    def body(cnt_ref, xs_ref, out_ref, pad_ref, stg_ref, send_sems,
             recv_sems, pad_ssem, pad_rsem):
        mx = lax.axis_index("x")
        my = lax.axis_index("y")
        mz = lax.axis_index("z")
        partner = (1 - mx, my, mz)

        barrier_sem = pltpu.get_barrier_semaphore()
        pl.semaphore_signal(
            barrier_sem,
            inc=1,
            device_id=partner,
            device_id_type=pl.DeviceIdType.MESH,
        )
        pl.semaphore_wait(barrier_sem, 1)

        c0v = cnt_ref[0]
        is0 = mx == 0
        cs = jnp.where(is0, t - c0v, c0v)
        n = (cs + CH - 1) // CH

        def off_at(k):
            return jnp.where(is0, k * CH, t - (k + 1) * CH)

        def stage(k):
            off = off_at(k)
            q = CH + c0v + off - jnp.where(is0, 0, t)
            qa = pl.multiple_of((q // 8) * 8, 8)
            w = xs_ref[pl.ds(qa, CH + 8), :]
            s = jnp.where(q == qa, 0, CH + 8 - (q - qa))
            w = pltpu.roll(w, s, axis=0)
            stg_ref[pl.ds(off, CH), :] = w[:CH, :]

        def full_chunk(k):
            off = off_at(k)
            return pltpu.make_async_remote_copy(
                src_ref=stg_ref.at[pl.ds(off, CH), :],
                dst_ref=out_ref.at[pl.ds(off, CH), :],
                send_sem=send_sems.at[k],
                recv_sem=recv_sems.at[k],
                device_id=partner,
                device_id_type=pl.DeviceIdType.MESH,
            )

        def pad_chunk():
            off = off_at(n - 1)
            return pltpu.make_async_remote_copy(
                src_ref=stg_ref.at[pl.ds(off, CH), :],
                dst_ref=pad_ref,
                send_sem=pad_ssem,
                recv_sem=pad_rsem,
                device_id=partner,
                device_id_type=pl.DeviceIdType.MESH,
            )

        for k in range(maxc):
            @pl.when(k < n - 1)
            def _(k=k):
                stage(k)
                full_chunk(k).start()

        @pl.when(n > 0)
        def _():
            stage(n - 1)
            pad_chunk().start()

        for k in range(maxc):
            @pl.when(k < maxc - n)
            def _(k=k):
                off = off_at(k)
                out_ref[pl.ds(off, CH), :] = xs_ref[pl.ds(CH + off, CH), :]

        for k in range(maxc):
            @pl.when(k < n - 1)
            def _(k=k):
                full_chunk(k).wait_recv()

        @pl.when(n > 0)
        def _():
            pad_chunk().wait_recv()
            qb = jnp.where(is0, t - n * CH, (n - 1) * CH)
            rows = qb + lax.broadcasted_iota(jnp.int32, (CH, 1), 0)
            s = 1 - 2 * mx
            own_mask = (s * rows) < (s * c0v + mx)
            out_ref[pl.ds(qb, CH), :] = jnp.where(
                own_mask, xs_ref[pl.ds(CH + qb, CH), :], pad_ref[:, :]
            )

        for k in range(maxc):
            @pl.when(k < n - 1)
            def _(k=k):
                full_chunk(k).wait_send()

        @pl.when(n > 0)
        def _():
            pad_chunk().wait_send()

    return pl.pallas_call(
        body,
        out_shape=jax.ShapeDtypeStruct((t, d), x.dtype),
        in_specs=[
            pl.BlockSpec(memory_space=pltpu.SMEM),
            pl.BlockSpec(memory_space=pltpu.VMEM),
        ],
        out_specs=pl.BlockSpec(memory_space=pltpu.VMEM),
        scratch_shapes=[
            pltpu.VMEM((CH, d), x.dtype),
            pltpu.VMEM((t, d), x.dtype),
            pltpu.SemaphoreType.DMA((maxc,)),
            pltpu.SemaphoreType.DMA((maxc,)),
            pltpu.SemaphoreType.DMA,
            pltpu.SemaphoreType.DMA,
        ],
        compiler_params=pltpu.CompilerParams(collective_id=0),
    )(cnt, xs_pad)


# device time: 27768 ns/iter; 9.3380x vs baseline; 9.3380x over previous
import jax
import jax.numpy as jnp
from jax import lax
from jax.experimental import pallas as pl
from jax.experimental.pallas import tpu as pltpu

CH = 64


def kernel(x, dest):
    t, d = x.shape
    maxc = t // CH

    iota = jnp.arange(t, dtype=jnp.int32)
    d0 = dest == 0
    cum0 = jnp.cumsum(d0.astype(jnp.int32))
    c0 = cum0[-1]
    rank = jnp.where(d0, cum0 - 1, iota + c0 - cum0)
    rank2 = rank + (t - c0)
    rank2 = jnp.where(rank2 >= t, rank2 - t, rank2)
    cols = iota[None, :]
    rows = iota[:, None]
    inv = jnp.sum(jnp.where(rank[:, None] == cols, rows, 0), axis=0)
    inv2 = jnp.sum(jnp.where(rank2[:, None] == cols, rows, 0), axis=0)
    both = jnp.concatenate([inv, inv2]).astype(jnp.int32)
    big = x[both]
    cnt = jnp.reshape(c0, (1,))

    def body(cnt_ref, big_ref, out_ref, pad_ref, send_sems, recv_sems,
             pad_ssem, pad_rsem):
        mx = lax.axis_index("x")
        my = lax.axis_index("y")
        mz = lax.axis_index("z")
        partner = (1 - mx, my, mz)

        barrier_sem = pltpu.get_barrier_semaphore()
        pl.semaphore_signal(
            barrier_sem,
            inc=1,
            device_id=partner,
            device_id_type=pl.DeviceIdType.MESH,
        )
        pl.semaphore_wait(barrier_sem, 1)

        c0v = cnt_ref[0]
        is0 = mx == 0
        cs = jnp.where(is0, t - c0v, c0v)
        n = (cs + CH - 1) // CH

        def off_at(k):
            return jnp.where(is0, k * CH, t - (k + 1) * CH)

        def full_chunk(k):
            off = off_at(k)
            return pltpu.make_async_remote_copy(
                src_ref=big_ref.at[pl.ds(t + off, CH), :],
                dst_ref=out_ref.at[pl.ds(off, CH), :],
                send_sem=send_sems.at[k],
                recv_sem=recv_sems.at[k],
                device_id=partner,
                device_id_type=pl.DeviceIdType.MESH,
            )

        def pad_chunk():
            off = off_at(n - 1)
            return pltpu.make_async_remote_copy(
                src_ref=big_ref.at[pl.ds(t + off, CH), :],
                dst_ref=pad_ref,
                send_sem=pad_ssem,
                recv_sem=pad_rsem,
                device_id=partner,
                device_id_type=pl.DeviceIdType.MESH,
            )

        for k in range(maxc):
            @pl.when(k < n - 1)
            def _(k=k):
                full_chunk(k).start()

        @pl.when(n > 0)
        def _():
            pad_chunk().start()

        for k in range(maxc):
            @pl.when(k < maxc - n)
            def _(k=k):
                off = off_at(k)
                out_ref[pl.ds(off, CH), :] = big_ref[pl.ds(off, CH), :]

        for k in range(maxc):
            @pl.when(k < n - 1)
            def _(k=k):
                full_chunk(k).wait_recv()

        @pl.when(n > 0)
        def _():
            pad_chunk().wait_recv()
            qb = jnp.where(is0, t - n * CH, (n - 1) * CH)
            rows = qb + lax.broadcasted_iota(jnp.int32, (CH, 1), 0)
            s = 1 - 2 * mx
            own_mask = (s * rows) < (s * c0v + mx)
            out_ref[pl.ds(qb, CH), :] = jnp.where(
                own_mask, big_ref[pl.ds(qb, CH), :], pad_ref[:, :]
            )

        for k in range(maxc):
            @pl.when(k < n - 1)
            def _(k=k):
                full_chunk(k).wait_send()

        @pl.when(n > 0)
        def _():
            pad_chunk().wait_send()

    return pl.pallas_call(
        body,
        out_shape=jax.ShapeDtypeStruct((t, d), x.dtype),
        in_specs=[
            pl.BlockSpec(memory_space=pltpu.SMEM),
            pl.BlockSpec(memory_space=pltpu.VMEM),
        ],
        out_specs=pl.BlockSpec(memory_space=pltpu.VMEM),
        scratch_shapes=[
            pltpu.VMEM((CH, d), x.dtype),
            pltpu.SemaphoreType.DMA((maxc,)),
            pltpu.SemaphoreType.DMA((maxc,)),
            pltpu.SemaphoreType.DMA,
            pltpu.SemaphoreType.DMA,
        ],
        compiler_params=pltpu.CompilerParams(collective_id=0),
    )(cnt, big)
